# baseline (speedup 1.0000x reference)
import jax
import jax.numpy as jnp
from jax.experimental import pallas as pl
from jax.experimental.pallas import tpu as pltpu

_KH = _KW = 3
_PAD = 1
_BN_EPS = 1e-5


def _conv_stats_kernel(x_ref, w_ref, y_ref, stats_ref, xpad_ref, col_ref):
    _, HW, Cin = x_ref.shape
    Hp, Wp, _ = xpad_ref.shape
    H, W = Hp - 2 * _PAD, Wp - 2 * _PAD

    xpad_ref[...] = jnp.zeros(xpad_ref.shape, xpad_ref.dtype)
    xpad_ref[_PAD:_PAD + H, _PAD:_PAD + W, :] = x_ref[0].reshape(H, W, Cin)

    for dy in range(_KH):
        for dx in range(_KW):
            t = dy * _KW + dx
            col_ref[:, t * Cin:(t + 1) * Cin] = (
                xpad_ref[dy:dy + H, dx:dx + W, :].reshape(HW, Cin))

    y = jnp.dot(col_ref[...], w_ref[...], preferred_element_type=jnp.float32)
    y_ref[...] = y.astype(jnp.bfloat16)

    s1 = jnp.sum(y, axis=0, keepdims=True)
    s2 = jnp.sum(y * y, axis=0, keepdims=True)
    stats_ref[...] = jnp.concatenate([s1, s2], axis=0)[None]


def _bn_relu_kernel(y_ref, ab_ref, o_ref):
    scale = ab_ref[0:1, :]
    shift = ab_ref[1:2, :]
    y = y_ref[...].astype(jnp.float32)
    o_ref[...] = jnp.maximum(y * scale + shift, 0.0)


def kernel(x_nchw, weight, bias):
    del bias
    N, Cin, H, W = x_nchw.shape
    Cout = weight.shape[0]
    K = _KH * _KW * Cin
    HW = H * W
    rows = N * HW

    x_rows = jnp.transpose(x_nchw.reshape(N, Cin, HW),
                           (0, 2, 1)).astype(jnp.bfloat16)
    w_flat = jnp.transpose(weight, (2, 3, 1, 0)).reshape(K, Cout)
    w_flat = w_flat.astype(jnp.bfloat16)

    y_conv, part_stats = pl.pallas_call(
        _conv_stats_kernel,
        out_shape=(jax.ShapeDtypeStruct((rows, Cout), jnp.bfloat16),
                   jax.ShapeDtypeStruct((N, 2, Cout), jnp.float32)),
        grid=(N,),
        in_specs=[
            pl.BlockSpec((1, HW, Cin), lambda i: (i, 0, 0)),
            pl.BlockSpec((K, Cout), lambda i: (0, 0)),
        ],
        out_specs=[
            pl.BlockSpec((HW, Cout), lambda i: (i, 0)),
            pl.BlockSpec((1, 2, Cout), lambda i: (i, 0, 0)),
        ],
        scratch_shapes=[
            pltpu.VMEM((H + 2 * _PAD, W + 2 * _PAD, Cin), jnp.bfloat16),
            pltpu.VMEM((HW, K), jnp.bfloat16),
        ],
        compiler_params=pltpu.CompilerParams(
            dimension_semantics=("parallel",)),
    )(x_rows, w_flat)

    sums = jnp.sum(part_stats, axis=0)
    mean = sums[0] / rows
    var = sums[1] / rows - mean * mean
    inv_std = jax.lax.rsqrt(var + _BN_EPS)
    ab = jnp.stack([inv_std, -mean * inv_std], axis=0)

    tile = 2 * HW
    out_flat = pl.pallas_call(
        _bn_relu_kernel,
        out_shape=jax.ShapeDtypeStruct((rows, Cout), jnp.float32),
        grid=(rows // tile,),
        in_specs=[
            pl.BlockSpec((tile, Cout), lambda i: (i, 0)),
            pl.BlockSpec((2, Cout), lambda i: (0, 0)),
        ],
        out_specs=pl.BlockSpec((tile, Cout), lambda i: (i, 0)),
        compiler_params=pltpu.CompilerParams(
            dimension_semantics=("parallel",)),
    )(y_conv, ab)

    out_nhwc = out_flat.reshape(N, H, W, Cout)
    return jnp.transpose(out_nhwc, (0, 3, 1, 2))


# speedup vs baseline: 1.5575x; 1.5575x over previous
import jax
import jax.numpy as jnp
from jax.experimental import pallas as pl
from jax.experimental.pallas import tpu as pltpu

_KH = _KW = 3
_PAD = 1
_BN_EPS = 1e-5


def _conv_stats_kernel(x_ref, w_ref, y_ref, stats_ref, xpad_ref, col_ref):
    _, HW, Cin = x_ref.shape
    Hp, Wp, _ = xpad_ref.shape
    H, W = Hp - 2 * _PAD, Wp - 2 * _PAD

    xpad_ref[...] = jnp.zeros(xpad_ref.shape, xpad_ref.dtype)
    xpad_ref[_PAD:_PAD + H, _PAD:_PAD + W, :] = x_ref[0].reshape(H, W, Cin)

    for dy in range(_KH):
        for dx in range(_KW):
            t = dy * _KW + dx
            col_ref[:, t * Cin:(t + 1) * Cin] = (
                xpad_ref[dy:dy + H, dx:dx + W, :].reshape(HW, Cin))

    y = jnp.dot(col_ref[...], w_ref[...], preferred_element_type=jnp.float32)
    y_ref[...] = y.astype(jnp.bfloat16)

    s1 = jnp.sum(y, axis=0, keepdims=True)
    s2 = jnp.sum(y * y, axis=0, keepdims=True)
    stats_ref[...] = jnp.concatenate([s1, s2], axis=0)[None]


def _bn_relu_kernel(y_ref, ab_ref, o_ref):
    scale = ab_ref[0:1, :]
    shift = ab_ref[1:2, :]
    y = y_ref[...].astype(jnp.float32)
    o_ref[...] = jnp.maximum(y * scale + shift, 0.0)


def kernel(x_nchw, weight, bias):
    del bias
    N, Cin, H, W = x_nchw.shape
    Cout = weight.shape[0]
    K = _KH * _KW * Cin
    HW = H * W
    rows = N * HW

    x_rows = jnp.transpose(x_nchw.reshape(N, Cin, HW), (0, 2, 1))
    w_flat = jnp.transpose(weight, (2, 3, 1, 0)).reshape(K, Cout)

    y_conv, part_stats = pl.pallas_call(
        _conv_stats_kernel,
        out_shape=(jax.ShapeDtypeStruct((rows, Cout), jnp.bfloat16),
                   jax.ShapeDtypeStruct((N, 2, Cout), jnp.float32)),
        grid=(N,),
        in_specs=[
            pl.BlockSpec((1, HW, Cin), lambda i: (i, 0, 0)),
            pl.BlockSpec((K, Cout), lambda i: (0, 0)),
        ],
        out_specs=[
            pl.BlockSpec((HW, Cout), lambda i: (i, 0)),
            pl.BlockSpec((1, 2, Cout), lambda i: (i, 0, 0)),
        ],
        scratch_shapes=[
            pltpu.VMEM((H + 2 * _PAD, W + 2 * _PAD, Cin), jnp.float32),
            pltpu.VMEM((HW, K), jnp.float32),
        ],
        compiler_params=pltpu.CompilerParams(
            dimension_semantics=("parallel",)),
    )(x_rows, w_flat)

    sums = jnp.sum(part_stats, axis=0)
    mean = sums[0] / rows
    var = sums[1] / rows - mean * mean
    inv_std = jax.lax.rsqrt(var + _BN_EPS)
    ab = jnp.stack([inv_std, -mean * inv_std], axis=0)

    tile = 2 * HW
    out_flat = pl.pallas_call(
        _bn_relu_kernel,
        out_shape=jax.ShapeDtypeStruct((rows, Cout), jnp.float32),
        grid=(rows // tile,),
        in_specs=[
            pl.BlockSpec((tile, Cout), lambda i: (i, 0)),
            pl.BlockSpec((2, Cout), lambda i: (0, 0)),
        ],
        out_specs=pl.BlockSpec((tile, Cout), lambda i: (i, 0)),
        compiler_params=pltpu.CompilerParams(
            dimension_semantics=("parallel",)),
    )(y_conv, ab)

    out_nhwc = out_flat.reshape(N, H, W, Cout)
    return jnp.transpose(out_nhwc, (0, 3, 1, 2))
